# baseline (speedup 1.0000x reference)
import functools

import jax
import jax.numpy as jnp
from jax import lax
from jax.experimental import pallas as pl
from jax.experimental.pallas import tpu as pltpu
from jax.experimental.pallas import tpu_sc as plsc

N = 10000
D_IN = 128
D_EDGE = 16
NS = 16
CHUNK = 128
G = 8
ACC_ROWS = 10240
ROWS_PER_TILE = ACC_ROWS // NS

_SCRATCH = [
    pltpu.VMEM_SHARED((ACC_ROWS, D_IN), jnp.float32),
    pltpu.VMEM((2 * G, CHUNK), jnp.int32),
    pltpu.VMEM((2 * G, CHUNK), jnp.int32),
    pltpu.VMEM((2, CHUNK, D_IN), jnp.float32),
    pltpu.VMEM((2, CHUNK * D_EDGE), jnp.float32),
    pltpu.SemaphoreType.DMA,
    pltpu.SemaphoreType.DMA,
    pltpu.SemaphoreType.DMA,
    pltpu.SemaphoreType.DMA,
]


def _mesh():
    return plsc.VectorSubcoreMesh(core_axis_name="c", subcore_axis_name="s",
                                  num_cores=1)


def _zero_acc(sid, rows, acc):
    def zrow(r, _):
        for bb in range(2):
            for cc in range(D_IN // 16):
                rows[bb, r, pl.ds(cc * 16, 16)] = jnp.zeros((16,),
                                                            jnp.float32)
        return 0
    lax.fori_loop(0, CHUNK, zrow, 0)
    zbase = sid * ROWS_PER_TILE
    for k in range(ROWS_PER_TILE // CHUNK):
        pltpu.sync_copy(rows.at[0], acc.at[pl.ds(zbase + k * CHUNK, CHUNK)])
    plsc.subcore_barrier()
    return zbase


def _copy_out(zbase, acc, rows, out_hbm):
    plsc.subcore_barrier()
    for k in range(ROWS_PER_TILE // CHUNK):
        sl = pl.ds(zbase + k * CHUNK, CHUNK)
        pltpu.sync_copy(acc.at[sl], rows.at[0])
        pltpu.sync_copy(rows.at[0], out_hbm.at[sl])


def _sc_gather_x(x, src2d, dst2d):
    n_chunks = src2d.shape[0]
    per_tile = n_chunks // NS
    n_groups = per_tile // G

    @functools.partial(
        pl.kernel,
        out_type=jax.ShapeDtypeStruct((ACC_ROWS, D_IN), jnp.float32),
        mesh=_mesh(),
        scratch_types=_SCRATCH,
    )
    def agg(x_hbm, src_hbm, dst_hbm, px_hbm,
            acc, srcb, dstb, rows, ecomp, g0, g1, s0, s1):
        sid = lax.axis_index("s")
        gsem = (g0, g1)
        ssem = (s0, s1)

        def drain_rows(b, sem):
            pltpu.make_async_copy(
                x_hbm.at[pl.ds(0, CHUNK)], rows.at[b], sem).wait()

        zbase = _zero_acc(sid, rows, acc)
        gtile = sid * per_tile

        def group_body(g, _):
            gb = lax.rem(g, 2)
            half = gb * G
            grow = gtile + g * G
            pltpu.sync_copy(src_hbm.at[pl.ds(grow, G)],
                            srcb.at[pl.ds(half, G)])
            pltpu.sync_copy(dst_hbm.at[pl.ds(grow, G)],
                            dstb.at[pl.ds(half, G)])

            @pl.when(g >= 1)
            def _():
                drain_rows(0, ssem[0])
            pltpu.async_copy(x_hbm.at[srcb.at[half]], rows.at[0], gsem[0])

            for k in range(G):
                b = k % 2
                j = g * G + k
                if k + 1 < G:
                    nb = (k + 1) % 2

                    @pl.when(j + 1 >= 2)
                    def _():
                        drain_rows(nb, ssem[nb])
                    pltpu.async_copy(x_hbm.at[srcb.at[half + k + 1]],
                                     rows.at[nb], gsem[nb])
                drain_rows(b, gsem[b])
                pltpu.async_copy(rows.at[b], acc.at[dstb.at[half + k]],
                                 ssem[b], add=True)
            return 0
        lax.fori_loop(0, n_groups, group_body, 0)
        drain_rows(0, ssem[0])
        drain_rows(1, ssem[1])

        _copy_out(zbase, acc, rows, px_hbm)

    return agg(x, src2d, dst2d)


def _sc_gather_e(dst2d, ea_flat):
    n_chunks = dst2d.shape[0]
    per_tile = n_chunks // NS
    n_groups = per_tile // G

    @functools.partial(
        pl.kernel,
        out_type=jax.ShapeDtypeStruct((ACC_ROWS, D_IN), jnp.float32),
        mesh=_mesh(),
        scratch_types=_SCRATCH,
    )
    def agg(dst_hbm, ea_hbm, pe_hbm,
            acc, srcb, dstb, rows, ecomp, g0, g1, s0, s1):
        sid = lax.axis_index("s")
        gsem = (g0, g1)
        ssem = (s0, s1)

        def drain_rows(b, sem):
            pltpu.make_async_copy(
                pe_hbm.at[pl.ds(0, CHUNK)], rows.at[b], sem).wait()

        def drain_ecomp(b, sem):
            pltpu.make_async_copy(
                ea_hbm.at[pl.ds(0, CHUNK * D_EDGE)], ecomp.at[b], sem).wait()

        zbase = _zero_acc(sid, rows, acc)
        gtile = sid * per_tile

        def group_body(g, _):
            gb = lax.rem(g, 2)
            half = gb * G
            grow = gtile + g * G
            pltpu.sync_copy(dst_hbm.at[pl.ds(grow, G)],
                            dstb.at[pl.ds(half, G)])

            ebase = grow * CHUNK * D_EDGE
            pltpu.async_copy(ea_hbm.at[pl.ds(ebase, CHUNK * D_EDGE)],
                             ecomp.at[0], gsem[0])

            for k in range(G):
                b = k % 2
                j = g * G + k
                if k + 1 < G:
                    nb = (k + 1) % 2
                    pltpu.async_copy(
                        ea_hbm.at[pl.ds(ebase + (k + 1) * CHUNK * D_EDGE,
                                        CHUNK * D_EDGE)],
                        ecomp.at[nb], gsem[nb])
                drain_ecomp(b, gsem[b])

                @pl.when(j >= 2)
                def _():
                    drain_rows(b, ssem[b])

                def expand(e, _):
                    rows[b, e, pl.ds(0, D_EDGE)] = ecomp[
                        b, pl.ds(e * D_EDGE, D_EDGE)]
                    return 0
                lax.fori_loop(0, CHUNK, expand, 0)
                pltpu.async_copy(rows.at[b], acc.at[dstb.at[half + k]],
                                 ssem[b], add=True)
            return 0
        lax.fori_loop(0, n_groups, group_body, 0)
        drain_rows(0, ssem[0])
        drain_rows(1, ssem[1])

        _copy_out(zbase, acc, rows, pe_hbm)

    return agg(dst2d, ea_flat)


def _tc_combine(px0, px1, pe, wxt, wet_pad):
    blk = 1024
    grid = ACC_ROWS // blk

    def body(a0, a1, e, wx, we, o):
        o[...] = (jnp.dot(a0[...] + a1[...], wx[...],
                          preferred_element_type=jnp.float32)
                  + jnp.dot(e[...], we[...],
                            preferred_element_type=jnp.float32))

    return pl.pallas_call(
        body,
        grid=(grid,),
        in_specs=[
            pl.BlockSpec((blk, D_IN), lambda i: (i, 0)),
            pl.BlockSpec((blk, D_IN), lambda i: (i, 0)),
            pl.BlockSpec((blk, D_IN), lambda i: (i, 0)),
            pl.BlockSpec((D_IN, D_IN), lambda i: (0, 0)),
            pl.BlockSpec((D_IN, D_IN), lambda i: (0, 0)),
        ],
        out_specs=pl.BlockSpec((blk, D_IN), lambda i: (i, 0)),
        out_shape=jax.ShapeDtypeStruct((ACC_ROWS, D_IN), jnp.float32),
    )(px0, px1, pe, wxt, wet_pad)


@jax.jit
def kernel(x, edge_index, edge_attr, W):
    E = edge_index.shape[1]
    unit = 2 * NS * CHUNK * G
    e_pad = -(-E // unit) * unit
    pad = e_pad - E

    src = jnp.concatenate([edge_index[0], jnp.zeros((pad,), jnp.int32)])
    dst = jnp.concatenate([edge_index[1], jnp.full((pad,), N, jnp.int32)])
    src2d = src.reshape(-1, CHUNK)
    dst2d = dst.reshape(-1, CHUNK)
    ea_flat = jnp.concatenate(
        [edge_attr.reshape(-1), jnp.zeros((pad * D_EDGE,), jnp.float32)])

    h = src2d.shape[0] // 2
    px0 = _sc_gather_x(x, src2d[:h], dst2d[:h])
    px1 = _sc_gather_x(x, src2d[h:], dst2d[h:])
    pe = _sc_gather_e(dst2d, ea_flat)

    wxt = W[:, :D_IN].T
    wet_pad = jnp.concatenate(
        [W[:, D_IN:].T, jnp.zeros((D_IN - D_EDGE, D_IN), jnp.float32)])
    out = _tc_combine(px0, px1, pe, wxt, wet_pad)
    return out[:N]
